# baseline (speedup 1.0000x reference)
import dataclasses
import functools

import jax
import jax.numpy as jnp
from jax import lax
from jax.experimental import pallas as pl
from jax.experimental.pallas import tpu as pltpu
from jax.experimental.pallas import tpu_sc as plsc

_B = 16384
_K = 1024
_R = 8
_NW = 32
_DEPTH = 4
_NB = _B // (_NW * _R)


def _sc_permute(x, perm):
    mesh = plsc.VectorSubcoreMesh(core_axis_name="c", subcore_axis_name="s")
    cp = pltpu.CompilerParams()
    if "needs_layout_passes" in pltpu.CompilerParams.__dataclass_fields__:
        cp = dataclasses.replace(cp, needs_layout_passes=False)

    scratch = [pltpu.VMEM((_K,), jnp.int32)]
    scratch += [pltpu.VMEM((_R, _K), jnp.float32) for _ in range(2 * _DEPTH)]
    scratch += [pltpu.SemaphoreType.DMA for _ in range(2 * _DEPTH)]

    @functools.partial(
        pl.kernel,
        mesh=mesh,
        out_type=jax.ShapeDtypeStruct((_B, _K), jnp.float32),
        scratch_types=scratch,
        compiler_params=cp,
    )
    def run(x_hbm, perm_hbm, out_hbm, idx_v, *bufs_and_sems):
        ins = bufs_and_sems[:_DEPTH]
        outs = bufs_and_sems[_DEPTH:2 * _DEPTH]
        sis = bufs_and_sems[2 * _DEPTH:3 * _DEPTH]
        sos = bufs_and_sems[3 * _DEPTH:4 * _DEPTH]

        wid = lax.axis_index("s") * 2 + lax.axis_index("c")
        base = wid * (_NB * _R)
        pltpu.sync_copy(perm_hbm, idx_v)

        def rows(g):
            return pl.ds(base + g * _R, _R)

        def compute(in_v, out_v):
            @pl.loop(0, _K // 16)
            def _(kc):
                col = idx_v[pl.ds(kc * 16, 16)]

                @plsc.parallel_loop(0, _R, 1, unroll=_R)
                def _(r):
                    row = jnp.full((16,), r, jnp.int32)
                    out_v[r, pl.ds(kc * 16, 16)] = plsc.load_gather(
                        in_v, [row, col]
                    )

        for b in range(_DEPTH):
            pltpu.async_copy(x_hbm.at[rows(b)], ins[b], sis[b])

        @pl.loop(0, _NB // _DEPTH)
        def _(it):
            g = it * _DEPTH
            for b in range(_DEPTH):
                pltpu.make_async_copy(x_hbm.at[rows(0)], ins[b], sis[b]).wait()

                @pl.when(it > 0)
                def _():
                    pltpu.make_async_copy(
                        outs[b], out_hbm.at[rows(0)], sos[b]
                    ).wait()

                compute(ins[b], outs[b])
                pltpu.async_copy(outs[b], out_hbm.at[rows(g + b)], sos[b])

                @pl.when(it < _NB // _DEPTH - 1)
                def _():
                    pltpu.async_copy(
                        x_hbm.at[rows(g + b + _DEPTH)], ins[b], sis[b]
                    )

        for b in range(_DEPTH):
            pltpu.make_async_copy(outs[b], out_hbm.at[rows(0)], sos[b]).wait()

    return run(x, perm)


def kernel(x, matrix):
    perm = jnp.transpose(matrix).reshape(_K).astype(jnp.int32)
    out = _sc_permute(x, perm)
    return out.reshape(_B, 32, 32)
